# baseline (speedup 1.0000x reference)
import jax
import jax.numpy as jnp
from jax import lax
from jax.experimental import pallas as pl
from jax.experimental.pallas import tpu as pltpu
from jax.experimental.pallas import tpu_sc as plsc

L = 50
D = 64
NC = 2
NS = 16
NW = NC * NS
K = 8
RPB = K * L
GPB = RPB // 16
SSTR = 512

B_A = 4096
B_N = 16384
VHALF = 500000
PAD = 128


def _pool_body(a_ids, p_ids, n_ids, table_hbm, a_out, p_out, n_out,
               idx_v, qidx_v, off_v, rows_v, out_v, gsem):
    c = lax.axis_index("c")
    s = lax.axis_index("s")
    wid = s * NC + c

    def process(ids_hbm, out_hbm, seg_per_w):
        nb = seg_per_w // K
        seg0 = wid * seg_per_w
        id0 = seg0 * L

        def issue(block, slot):
            base = id0 + block * RPB
            pltpu.sync_copy(ids_hbm.at[pl.ds(base, SSTR)],
                            idx_v.at[pl.ds(slot * SSTR, SSTR)])
            for j in range(GPB):
                idv = idx_v[pl.ds(slot * SSTR + j * 16, 16)]
                qidx_v[pl.ds(slot * SSTR + j * 16, 16)] = (
                    lax.shift_right_logical(idv, 1))
                off_v[pl.ds(slot * SSTR + j * 16, 16)] = lax.mul(
                    lax.rem(idv, 2), jnp.int32(64))
            for m in range(GPB):
                qv = qidx_v[pl.ds(slot * SSTR + m * 16, 16)]
                pltpu.async_copy(
                    table_hbm.at[qv],
                    rows_v.at[pl.ds(slot * RPB + m * 16, 16)],
                    gsem.at[slot])

        issue(0, 0)

        zidx = jnp.zeros((16,), jnp.int32)

        def step(g, carry):
            slot = lax.rem(g, 2)

            @pl.when(g + 1 < nb)
            def _():
                issue(g + 1, 1 - slot)

            for m in range(GPB):
                pltpu.make_async_copy(
                    table_hbm.at[zidx],
                    rows_v.at[pl.ds(slot * RPB + m * 16, 16)],
                    gsem.at[slot]).wait()

            def seg_body(k, carry2):
                srow = slot * RPB + k * L
                obase = slot * SSTR + k * L
                ogroups = [off_v[pl.ds(obase + j * 16, 16)]
                           for j in range((L + 15) // 16)]
                offs = [ogroups[r // 16][r % 16] for r in range(L)]
                accs = [rows_v[srow, pl.ds(offs[0] + d * 16, 16)]
                        for d in range(4)]
                for r in range(1, L):
                    for d in range(4):
                        accs[d] = accs[d] + rows_v[
                            srow + r, pl.ds(offs[r] + d * 16, 16)]
                m = [a * jnp.float32(1.0 / L) for a in accs]
                ssv = m[0] * m[0] + m[1] * m[1] + m[2] * m[2] + m[3] * m[3]
                lane = lax.iota(jnp.int32, 16)
                dn = lax.GatherDimensionNumbers(offset_dims=(),
                                                collapsed_slice_dims=(0,),
                                                start_index_map=(0,))
                sv = ssv
                for sh in (8, 4, 2, 1):
                    perm = (lane ^ sh)[:, None]
                    sv = sv + lax.gather(
                        sv, perm, dn, slice_sizes=(1,),
                        mode=lax.GatherScatterMode.PROMISE_IN_BOUNDS)
                ii = lax.bitcast_convert_type(sv, jnp.int32)
                yi = jnp.int32(0x5F3759DF) - lax.shift_right_arithmetic(ii, 1)
                y = lax.bitcast_convert_type(yi, jnp.float32)
                for _ in range(3):
                    y = y * (jnp.float32(1.5) - jnp.float32(0.5) * sv * y * y)
                y = jnp.minimum(y, jnp.float32(1e12))
                for d in range(4):
                    out_v[slot * K + k, pl.ds(d * 16, 16)] = m[d] * y
                return carry2

            lax.fori_loop(0, K, seg_body, 0)
            pltpu.sync_copy(out_v.at[pl.ds(slot * K, K)],
                            out_hbm.at[pl.ds(pl.multiple_of(seg0 + g * K, 8),
                                             K)])
            return carry

        lax.fori_loop(0, nb, step, 0)

    process(a_ids, a_out, B_A // NW)
    process(p_ids, p_out, B_A // NW)
    process(n_ids, n_out, B_N // NW)


@jax.jit
def _pooled_normalized(a_ids, p_ids, n_ids, table2):
    run = pl.kernel(
        _pool_body,
        out_type=(
            jax.ShapeDtypeStruct((B_A, D), jnp.float32),
            jax.ShapeDtypeStruct((B_A, D), jnp.float32),
            jax.ShapeDtypeStruct((B_N, D), jnp.float32),
        ),
        mesh=plsc.VectorSubcoreMesh(core_axis_name="c", subcore_axis_name="s",
                                    num_cores=NC, num_subcores=NS),
        scratch_types=[
            pltpu.VMEM((2 * SSTR,), jnp.int32),
            pltpu.VMEM((2 * SSTR,), jnp.int32),
            pltpu.VMEM((2 * SSTR,), jnp.int32),
            pltpu.VMEM((2 * RPB, 2 * D), jnp.float32),
            pltpu.VMEM((2 * K, D), jnp.float32),
            pltpu.SemaphoreType.DMA((2,)),
        ],
        compiler_params=pltpu.CompilerParams(use_tc_tiling_on_sc=True),
    )
    return run(a_ids, p_ids, n_ids, table2)


def _flat_padded(ids):
    flat = ids.astype(jnp.int32).reshape(-1)
    return jnp.concatenate([flat, jnp.zeros((PAD,), jnp.int32)])


def kernel(anchor_input_ids, positive_input_ids, negative_input_ids,
           embedding_weight):
    return _pooled_normalized(
        _flat_padded(anchor_input_ids),
        _flat_padded(positive_input_ids),
        _flat_padded(negative_input_ids),
        embedding_weight.reshape(VHALF, 2 * D),
    )


# speedup vs baseline: 1.1633x; 1.1633x over previous
import jax
import jax.numpy as jnp
from jax import lax
from jax.experimental import pallas as pl
from jax.experimental.pallas import tpu as pltpu
from jax.experimental.pallas import tpu_sc as plsc

L = 50
D = 64
NC = 2
NS = 16
NW = NC * NS
K = 8
RPB = K * L
GPB = RPB // 16
SSTR = 512

B_A = 4096
B_N = 16384
VPAD = 1000000
PAD = 128


def _pool_body(a_ids, p_ids, n_ids, table_hbm, a_out, p_out, n_out,
               idx_v, rows_v, out_v, gsem):
    c = lax.axis_index("c")
    s = lax.axis_index("s")
    wid = s * NC + c

    def process(ids_hbm, out_hbm, seg_per_w):
        nb = seg_per_w // K
        seg0 = wid * seg_per_w
        id0 = seg0 * L

        def issue(block, slot):
            base = id0 + block * RPB
            pltpu.sync_copy(ids_hbm.at[pl.ds(base, SSTR)],
                            idx_v.at[pl.ds(slot * SSTR, SSTR)])
            for m in range(GPB):
                qv = idx_v[pl.ds(slot * SSTR + m * 16, 16)]
                pltpu.async_copy(
                    table_hbm.at[qv],
                    rows_v.at[pl.ds(slot * RPB + m * 16, 16)],
                    gsem.at[slot])

        issue(0, 0)

        zidx = jnp.zeros((16,), jnp.int32)

        def step(g, carry):
            slot = lax.rem(g, 2)

            @pl.when(g + 1 < nb)
            def _():
                issue(g + 1, 1 - slot)

            for m in range(GPB):
                pltpu.make_async_copy(
                    table_hbm.at[zidx],
                    rows_v.at[pl.ds(slot * RPB + m * 16, 16)],
                    gsem.at[slot]).wait()

            def seg_body(k, carry2):
                srow = slot * RPB + k * L
                accs = [rows_v[srow, pl.ds(d * 16, 16)] for d in range(4)]
                for r in range(1, L):
                    for d in range(4):
                        accs[d] = accs[d] + rows_v[
                            srow + r, pl.ds(d * 16, 16)]
                m = [a * jnp.float32(1.0 / L) for a in accs]
                ssv = m[0] * m[0] + m[1] * m[1] + m[2] * m[2] + m[3] * m[3]
                lane = lax.iota(jnp.int32, 16)
                dn = lax.GatherDimensionNumbers(offset_dims=(),
                                                collapsed_slice_dims=(0,),
                                                start_index_map=(0,))
                sv = ssv
                for sh in (8, 4, 2, 1):
                    perm = (lane ^ sh)[:, None]
                    sv = sv + lax.gather(
                        sv, perm, dn, slice_sizes=(1,),
                        mode=lax.GatherScatterMode.PROMISE_IN_BOUNDS)
                ii = lax.bitcast_convert_type(sv, jnp.int32)
                yi = jnp.int32(0x5F3759DF) - lax.shift_right_arithmetic(ii, 1)
                y = lax.bitcast_convert_type(yi, jnp.float32)
                for _ in range(3):
                    y = y * (jnp.float32(1.5) - jnp.float32(0.5) * sv * y * y)
                y = jnp.minimum(y, jnp.float32(1e12))
                for d in range(4):
                    out_v[slot * K + k, pl.ds(d * 16, 16)] = m[d] * y
                return carry2

            lax.fori_loop(0, K, seg_body, 0)
            pltpu.sync_copy(out_v.at[pl.ds(slot * K, K)],
                            out_hbm.at[pl.ds(pl.multiple_of(seg0 + g * K, 8),
                                             K)])
            return carry

        lax.fori_loop(0, nb, step, 0)

    process(a_ids, a_out, B_A // NW)
    process(p_ids, p_out, B_A // NW)
    process(n_ids, n_out, B_N // NW)


@jax.jit
def _pooled_normalized(a_ids, p_ids, n_ids, table2):
    run = pl.kernel(
        _pool_body,
        out_type=(
            jax.ShapeDtypeStruct((B_A, D), jnp.float32),
            jax.ShapeDtypeStruct((B_A, D), jnp.float32),
            jax.ShapeDtypeStruct((B_N, D), jnp.float32),
        ),
        mesh=plsc.VectorSubcoreMesh(core_axis_name="c", subcore_axis_name="s",
                                    num_cores=NC, num_subcores=NS),
        scratch_types=[
            pltpu.VMEM((2 * SSTR,), jnp.int32),
            pltpu.VMEM((2 * RPB, 2 * D), jnp.float32),
            pltpu.VMEM((2 * K, D), jnp.float32),
            pltpu.SemaphoreType.DMA((2,)),
        ],
        compiler_params=pltpu.CompilerParams(use_tc_tiling_on_sc=True),
    )
    return run(a_ids, p_ids, n_ids, table2)


def _flat_padded(ids):
    flat = ids.astype(jnp.int32).reshape(-1)
    return jnp.concatenate([flat, jnp.zeros((PAD,), jnp.int32)])


def kernel(anchor_input_ids, positive_input_ids, negative_input_ids,
           embedding_weight):
    return _pooled_normalized(
        _flat_padded(anchor_input_ids),
        _flat_padded(positive_input_ids),
        _flat_padded(negative_input_ids),
        jnp.pad(embedding_weight, ((0, 0), (0, D))),
    )
